# speedup vs baseline: 2.5943x; 2.5943x over previous
import jax
import jax.numpy as jnp
from jax.experimental import pallas as pl

EOS_ID = 2
HARD_THRED = 1.0
LOSS_WEIGHT = 1.0
B, L, V = 32, 64, 32000
ROW_BLOCK = 64


def _lse_kernel(g_ref, out_ref):
    x = g_ref[...]
    m = jnp.max(x, axis=-1, keepdims=True)
    s = jnp.sum(jnp.exp(x - m), axis=-1, keepdims=True)
    out_ref[...] = -jnp.log(s)


def _loss_kernel(tmp1_ref, ul_ref, ut_ref, times_ref, tok_ref, out_ref):
    tok = tok_ref[...]
    is_eos = (tok == EOS_ID).astype(jnp.int32)
    idx = jax.lax.broadcasted_iota(jnp.int32, (B, L), 1)
    first_eos = jnp.min(jnp.where(is_eos == 1, idx, L), axis=1, keepdims=True)
    mask = idx <= first_eos
    maskf = mask.astype(jnp.float32)
    mask_sum = jnp.sum(maskf, axis=1, keepdims=True)

    tmp1 = jnp.where(mask, tmp1_ref[...], -jnp.inf)
    tmp1 = jnp.where(tmp1 == 0.0, -jnp.inf, tmp1)

    ti = tmp1[:, :, None]
    tj = tmp1[:, None, :]
    ii = jax.lax.broadcasted_iota(jnp.int32, (B, L, L), 1)
    jj = jax.lax.broadcasted_iota(jnp.int32, (B, L, L), 2)
    beats = (tj > ti) | ((tj == ti) & (jj < ii))
    rank = jnp.sum(beats.astype(jnp.float32), axis=2)
    sel = (rank < 5.0).astype(jnp.float32)

    logits = ul_ref[...] * maskf
    preds = jnp.sum(logits * sel, axis=1, keepdims=True) / mask_sum
    img_label = jnp.sum(ut_ref[...] * maskf, axis=1, keepdims=True) / L

    p_col = preds
    p_row = preds.reshape(1, B)
    il_col = img_label
    il_row = img_label.reshape(1, B)
    t = times_ref[...]
    dt = jnp.abs(t - t.reshape(B, 1))
    masks_time = ((dt < 0.12) & (dt > 0.0)).astype(jnp.float32)
    dlab = il_row - il_col
    masks = jnp.sign(dlab) * masks_time
    adl = jnp.abs(dlab)
    masks_hard = ((adl < HARD_THRED) & (adl > 0.0)).astype(jnp.float32) * masks_time
    rank_loss = masks_hard * jnp.maximum(-masks * (p_row - p_col), 0.0)
    loss = jnp.sum(rank_loss) / (jnp.sum(masks_hard) + 1e-08)
    out_ref[...] = jnp.reshape(loss * LOSS_WEIGHT, (1, 1))


def kernel(u_logits, u_target_ids, g_logits, times, u_tokens_ids):
    g2 = g_logits.reshape(B * L, V)
    tmp1 = pl.pallas_call(
        _lse_kernel,
        grid=(B * L // ROW_BLOCK,),
        in_specs=[pl.BlockSpec((ROW_BLOCK, V), lambda i: (i, 0))],
        out_specs=pl.BlockSpec((ROW_BLOCK, 1), lambda i: (i, 0)),
        out_shape=jax.ShapeDtypeStruct((B * L, 1), jnp.float32),
    )(g2)

    loss = pl.pallas_call(
        _loss_kernel,
        in_specs=[
            pl.BlockSpec((B, L), lambda: (0, 0)),
            pl.BlockSpec((B, L), lambda: (0, 0)),
            pl.BlockSpec((B, L), lambda: (0, 0)),
            pl.BlockSpec((1, B), lambda: (0, 0)),
            pl.BlockSpec((B, L), lambda: (0, 0)),
        ],
        out_specs=pl.BlockSpec((1, 1), lambda: (0, 0)),
        out_shape=jax.ShapeDtypeStruct((1, 1), jnp.float32),
    )(
        tmp1.reshape(B, L),
        u_logits.reshape(B, L),
        u_target_ids.reshape(B, L),
        times.reshape(1, B),
        u_tokens_ids,
    )
    return loss.reshape(())
